# Initial kernel scaffold; baseline (speedup 1.0000x reference)
import jax
import jax.numpy as jnp
from jax.experimental import pallas as pl


def kernel(cls, label_cls, pred_bboxes, label_target):
    raise NotImplementedError("write your pallas kernel here")


# speedup vs baseline: 2136.8779x; 2136.8779x over previous
import jax
import jax.numpy as jnp
from jax.experimental import pallas as pl

IOU_G = 3.0
N = 3125
NPAD = 3200
BR = 128


def _iou_prob_body(cls_ref, pb_ref, lt_ref, iou_ref, p_ref):
    c = cls_ref[0]
    x1 = pb_ref[0, 0:1, :]
    y1 = pb_ref[0, 1:2, :]
    x2 = pb_ref[0, 2:3, :]
    y2 = pb_ref[0, 3:4, :]
    tx1 = lt_ref[0, :, 0:1]
    ty1 = lt_ref[0, :, 1:2]
    tx2 = lt_ref[0, :, 2:3]
    ty2 = lt_ref[0, :, 3:4]
    xx1 = jnp.maximum(tx1, x1)
    yy1 = jnp.maximum(ty1, y1)
    xx2 = jnp.minimum(tx2, x2)
    yy2 = jnp.minimum(ty2, y2)
    ww = jnp.maximum(xx2 - xx1, 0.0)
    hh = jnp.maximum(yy2 - yy1, 0.0)
    area = (x2 - x1) * (y2 - y1)
    target_a = (tx2 - tx1) * (ty2 - ty1)
    inter = ww * hh
    iou_ref[0] = inter / (area + target_a - inter)
    p_ref[0] = jnp.exp(c)


def _pair_body(iou_c_ref, p_c_ref, cls_c_ref, lab_c_ref,
               iou_r_ref, p_r_ref, cls_r_ref, lab_r_ref,
               s1_ref, s2_ref, np_ref):
    rb = pl.program_id(1)

    iou_c = iou_c_ref[0]
    p_c = p_c_ref[0]
    cls_c = cls_c_ref[0]
    pos_c = lab_c_ref[0] > 0
    iou_r = iou_r_ref[0]
    p_r = p_r_ref[0]
    cls_r = cls_r_ref[0]
    pos_r = lab_r_ref[0] > 0

    idx_c = jax.lax.broadcasted_iota(jnp.int32, (1, NPAD), 1)
    idx_r = jax.lax.broadcasted_iota(jnp.int32, (BR, 1), 0) + rb * BR

    tie = idx_r < idx_c
    m = pos_r & pos_c

    o1 = (iou_r > iou_c) | ((iou_r == iou_c) & tie)
    w1 = jnp.exp(-IOU_G * (p_r - p_c))
    t1 = jnp.where(m & o1, w1, 0.0)

    o2 = (cls_r > cls_c) | ((cls_r == cls_c) & tie)
    w2 = jnp.exp(-IOU_G * (iou_r - iou_c))
    t2 = jnp.where(m & o2, w2, 0.0)

    part1 = jnp.sum(t1, axis=0, keepdims=True)
    part2 = jnp.sum(t2, axis=0, keepdims=True)

    @pl.when(rb == 0)
    def _init():
        s1_ref[0] = part1
        s2_ref[0] = part2
        np_ref[0] = jnp.where(pos_c, 1.0, 0.0)

    @pl.when(rb != 0)
    def _acc():
        s1_ref[0] += part1
        s2_ref[0] += part2


def kernel(cls, label_cls, pred_bboxes, label_target):
    b = label_cls.shape[0]
    pad = NPAD - N

    cls1 = jnp.pad(cls[:, :, 1], ((0, 0), (0, pad)))
    lab = jnp.pad(label_cls.astype(jnp.int32), ((0, 0), (0, pad)))
    pb = jnp.pad(pred_bboxes, ((0, 0), (0, 0), (0, pad)))
    lt = label_target.reshape(b, 1, 4)

    iou, p = pl.pallas_call(
        _iou_prob_body,
        grid=(b,),
        in_specs=[
            pl.BlockSpec((1, 1, NPAD), lambda i: (i, 0, 0)),
            pl.BlockSpec((1, 4, NPAD), lambda i: (i, 0, 0)),
            pl.BlockSpec((1, 1, 4), lambda i: (i, 0, 0)),
        ],
        out_specs=[
            pl.BlockSpec((1, 1, NPAD), lambda i: (i, 0, 0)),
            pl.BlockSpec((1, 1, NPAD), lambda i: (i, 0, 0)),
        ],
        out_shape=[
            jax.ShapeDtypeStruct((b, 1, NPAD), jnp.float32),
            jax.ShapeDtypeStruct((b, 1, NPAD), jnp.float32),
        ],
    )(cls1.reshape(b, 1, NPAD), pb, lt)

    iou_r = jnp.swapaxes(iou, 1, 2)
    p_r = jnp.swapaxes(p, 1, 2)
    cls_c = cls1.reshape(b, 1, NPAD)
    cls_r = cls1.reshape(b, NPAD, 1)
    lab_c = lab.reshape(b, 1, NPAD)
    lab_r = lab.reshape(b, NPAD, 1)

    nb = NPAD // BR
    col = pl.BlockSpec((1, 1, NPAD), lambda i, j: (i, 0, 0))
    row = pl.BlockSpec((1, BR, 1), lambda i, j: (i, j, 0))
    out = pl.BlockSpec((1, 1, NPAD), lambda i, j: (i, 0, 0))

    s1, s2, npos_f = pl.pallas_call(
        _pair_body,
        grid=(b, nb),
        in_specs=[col, col, col, col, row, row, row, row],
        out_specs=[out, out, out],
        out_shape=[
            jax.ShapeDtypeStruct((b, 1, NPAD), jnp.float32),
            jax.ShapeDtypeStruct((b, 1, NPAD), jnp.float32),
            jax.ShapeDtypeStruct((b, 1, NPAD), jnp.float32),
        ],
    )(iou, p, cls_c, lab_c, iou_r, p_r, cls_r, lab_r)

    sum1 = s1.sum(axis=(1, 2))
    sum2 = s2.sum(axis=(1, 2))
    num_pos = npos_f.sum(axis=(1, 2)).astype(jnp.int32)

    denom = (num_pos * (num_pos - 1) // 2).astype(jnp.float32)
    loss1 = sum1 / denom
    loss2 = sum2 / denom
    valid = (num_pos > 0) & ~jnp.isnan(loss1) & ~jnp.isnan(loss2)
    cnt = valid.sum().astype(jnp.float32)
    any_valid = valid.any()
    safe_cnt = jnp.where(any_valid, cnt, jnp.asarray(1.0, jnp.float32))
    zero = jnp.asarray(0.0, jnp.float32)
    final1 = jnp.where(any_valid, jnp.where(valid, loss1, 0.0).sum() / safe_cnt, zero)
    final2 = jnp.where(any_valid, jnp.where(valid, loss2, 0.0).sum() / safe_cnt, zero)
    return (final1, final2)


# speedup vs baseline: 2533.2040x; 1.1855x over previous
import jax
import jax.numpy as jnp
from jax.experimental import pallas as pl

IOU_G = 3.0
N = 3125
NPAD = 3200
BR = 128


def _iou_prob_body(cls_ref, pb_ref, lt_ref, iou_ref, p_ref):
    c = cls_ref[0]
    x1 = pb_ref[0, 0:1, :]
    y1 = pb_ref[0, 1:2, :]
    x2 = pb_ref[0, 2:3, :]
    y2 = pb_ref[0, 3:4, :]
    tx1 = lt_ref[0, :, 0:1]
    ty1 = lt_ref[0, :, 1:2]
    tx2 = lt_ref[0, :, 2:3]
    ty2 = lt_ref[0, :, 3:4]
    xx1 = jnp.maximum(tx1, x1)
    yy1 = jnp.maximum(ty1, y1)
    xx2 = jnp.minimum(tx2, x2)
    yy2 = jnp.minimum(ty2, y2)
    ww = jnp.maximum(xx2 - xx1, 0.0)
    hh = jnp.maximum(yy2 - yy1, 0.0)
    area = (x2 - x1) * (y2 - y1)
    target_a = (tx2 - tx1) * (ty2 - ty1)
    inter = ww * hh
    iou_ref[0] = inter / (area + target_a - inter)
    p_ref[0] = jnp.exp(c)


def _pair_body(iou_c_ref, p_c_ref, cls_c_ref, lab_c_ref,
               iou_r_ref, p_r_ref, cls_r_ref, lab_r_ref,
               s1_ref, s2_ref, np_ref):
    rb = pl.program_id(1)

    iou_c = iou_c_ref[0]
    p_c = p_c_ref[0]
    cls_c = cls_c_ref[0]
    pos_c = lab_c_ref[0] > 0
    iou_r = iou_r_ref[0]
    p_r = p_r_ref[0]
    cls_r = cls_r_ref[0]
    pos_r = lab_r_ref[0] > 0

    idx_c = jax.lax.broadcasted_iota(jnp.int32, (1, NPAD), 1)
    idx_r = jax.lax.broadcasted_iota(jnp.int32, (BR, 1), 0) + rb * BR

    tie = idx_r < idx_c
    m = pos_r & pos_c

    o1 = (iou_r > iou_c) | ((iou_r == iou_c) & tie)
    w1 = jnp.exp(-IOU_G * (p_r - p_c))
    t1 = jnp.where(m & o1, w1, 0.0)

    o2 = (cls_r > cls_c) | ((cls_r == cls_c) & tie)
    w2 = jnp.exp(-IOU_G * (iou_r - iou_c))
    t2 = jnp.where(m & o2, w2, 0.0)

    ones = jnp.ones((1, BR), jnp.float32)
    part1 = jax.lax.dot_general(ones, t1, (((1,), (0,)), ((), ())),
                                preferred_element_type=jnp.float32)
    part2 = jax.lax.dot_general(ones, t2, (((1,), (0,)), ((), ())),
                                preferred_element_type=jnp.float32)

    @pl.when(rb == 0)
    def _init():
        s1_ref[0] = part1
        s2_ref[0] = part2
        np_ref[0] = jnp.where(pos_c, 1.0, 0.0)

    @pl.when(rb != 0)
    def _acc():
        s1_ref[0] += part1
        s2_ref[0] += part2


def kernel(cls, label_cls, pred_bboxes, label_target):
    b = label_cls.shape[0]
    pad = NPAD - N

    cls1 = jnp.pad(cls[:, :, 1], ((0, 0), (0, pad)))
    lab = jnp.pad(label_cls.astype(jnp.int32), ((0, 0), (0, pad)))
    pb = jnp.pad(pred_bboxes, ((0, 0), (0, 0), (0, pad)))
    lt = label_target.reshape(b, 1, 4)

    iou, p = pl.pallas_call(
        _iou_prob_body,
        grid=(b,),
        in_specs=[
            pl.BlockSpec((1, 1, NPAD), lambda i: (i, 0, 0)),
            pl.BlockSpec((1, 4, NPAD), lambda i: (i, 0, 0)),
            pl.BlockSpec((1, 1, 4), lambda i: (i, 0, 0)),
        ],
        out_specs=[
            pl.BlockSpec((1, 1, NPAD), lambda i: (i, 0, 0)),
            pl.BlockSpec((1, 1, NPAD), lambda i: (i, 0, 0)),
        ],
        out_shape=[
            jax.ShapeDtypeStruct((b, 1, NPAD), jnp.float32),
            jax.ShapeDtypeStruct((b, 1, NPAD), jnp.float32),
        ],
    )(cls1.reshape(b, 1, NPAD), pb, lt)

    iou_r = jnp.swapaxes(iou, 1, 2)
    p_r = jnp.swapaxes(p, 1, 2)
    cls_c = cls1.reshape(b, 1, NPAD)
    cls_r = cls1.reshape(b, NPAD, 1)
    lab_c = lab.reshape(b, 1, NPAD)
    lab_r = lab.reshape(b, NPAD, 1)

    nb = NPAD // BR
    col = pl.BlockSpec((1, 1, NPAD), lambda i, j: (i, 0, 0))
    row = pl.BlockSpec((1, BR, 1), lambda i, j: (i, j, 0))
    out = pl.BlockSpec((1, 1, NPAD), lambda i, j: (i, 0, 0))

    s1, s2, npos_f = pl.pallas_call(
        _pair_body,
        grid=(b, nb),
        in_specs=[col, col, col, col, row, row, row, row],
        out_specs=[out, out, out],
        out_shape=[
            jax.ShapeDtypeStruct((b, 1, NPAD), jnp.float32),
            jax.ShapeDtypeStruct((b, 1, NPAD), jnp.float32),
            jax.ShapeDtypeStruct((b, 1, NPAD), jnp.float32),
        ],
    )(iou, p, cls_c, lab_c, iou_r, p_r, cls_r, lab_r)

    sum1 = s1.sum(axis=(1, 2))
    sum2 = s2.sum(axis=(1, 2))
    num_pos = npos_f.sum(axis=(1, 2)).astype(jnp.int32)

    denom = (num_pos * (num_pos - 1) // 2).astype(jnp.float32)
    loss1 = sum1 / denom
    loss2 = sum2 / denom
    valid = (num_pos > 0) & ~jnp.isnan(loss1) & ~jnp.isnan(loss2)
    cnt = valid.sum().astype(jnp.float32)
    any_valid = valid.any()
    safe_cnt = jnp.where(any_valid, cnt, jnp.asarray(1.0, jnp.float32))
    zero = jnp.asarray(0.0, jnp.float32)
    final1 = jnp.where(any_valid, jnp.where(valid, loss1, 0.0).sum() / safe_cnt, zero)
    final2 = jnp.where(any_valid, jnp.where(valid, loss2, 0.0).sum() / safe_cnt, zero)
    return (final1, final2)


# speedup vs baseline: 2685.6328x; 1.0602x over previous
import jax
import jax.numpy as jnp
from jax.experimental import pallas as pl

IOU_G = 3.0
LOG2E = 1.4426950408889634
N = 3125
NPAD = 3200
BR = 320


def _iou_prob_body(cls_ref, pb_ref, lt_ref, iou_ref, a1_ref, a2_ref):
    c = cls_ref[0]
    x1 = pb_ref[0, 0:1, :]
    y1 = pb_ref[0, 1:2, :]
    x2 = pb_ref[0, 2:3, :]
    y2 = pb_ref[0, 3:4, :]
    tx1 = lt_ref[0, :, 0:1]
    ty1 = lt_ref[0, :, 1:2]
    tx2 = lt_ref[0, :, 2:3]
    ty2 = lt_ref[0, :, 3:4]
    xx1 = jnp.maximum(tx1, x1)
    yy1 = jnp.maximum(ty1, y1)
    xx2 = jnp.minimum(tx2, x2)
    yy2 = jnp.minimum(ty2, y2)
    ww = jnp.maximum(xx2 - xx1, 0.0)
    hh = jnp.maximum(yy2 - yy1, 0.0)
    area = (x2 - x1) * (y2 - y1)
    target_a = (tx2 - tx1) * (ty2 - ty1)
    inter = ww * hh
    iou = inter / (area + target_a - inter)
    iou_ref[0] = iou
    a1_ref[0] = (-IOU_G * LOG2E) * jnp.exp(c)
    a2_ref[0] = (-IOU_G * LOG2E) * iou


def _pair_body(iou_c_ref, a1_c_ref, a2_c_ref, cls_c_ref, lab_c_ref,
               iou_r_ref, a1_r_ref, a2_r_ref, cls_r_ref, lab_r_ref,
               s1_ref, s2_ref, np_ref):
    rb = pl.program_id(1)

    iou_c = iou_c_ref[0]
    a1_c = a1_c_ref[0]
    a2_c = a2_c_ref[0]
    cls_c = cls_c_ref[0]
    pos_c = lab_c_ref[0] > 0
    iou_r = iou_r_ref[0]
    a1_r = a1_r_ref[0]
    a2_r = a2_r_ref[0]
    cls_r = cls_r_ref[0]
    pos_r = lab_r_ref[0] > 0

    idx_c = jax.lax.broadcasted_iota(jnp.int32, (1, NPAD), 1)
    idx_r = jax.lax.broadcasted_iota(jnp.int32, (BR, 1), 0) + rb * BR

    tie = idx_r < idx_c
    m = pos_r & pos_c

    o1 = (iou_r > iou_c) | ((iou_r == iou_c) & tie)
    w1 = jnp.exp2(a1_r - a1_c)
    t1 = jnp.where(m & o1, w1, 0.0)

    o2 = (cls_r > cls_c) | ((cls_r == cls_c) & tie)
    w2 = jnp.exp2(a2_r - a2_c)
    t2 = jnp.where(m & o2, w2, 0.0)

    ones = jnp.ones((1, BR), jnp.float32)
    part1 = jax.lax.dot_general(ones, t1, (((1,), (0,)), ((), ())),
                                preferred_element_type=jnp.float32)
    part2 = jax.lax.dot_general(ones, t2, (((1,), (0,)), ((), ())),
                                preferred_element_type=jnp.float32)

    @pl.when(rb == 0)
    def _init():
        s1_ref[0] = part1
        s2_ref[0] = part2
        np_ref[0] = jnp.where(pos_c, 1.0, 0.0)

    @pl.when(rb != 0)
    def _acc():
        s1_ref[0] += part1
        s2_ref[0] += part2


def kernel(cls, label_cls, pred_bboxes, label_target):
    b = label_cls.shape[0]
    pad = NPAD - N

    cls1 = jnp.pad(cls[:, :, 1], ((0, 0), (0, pad)))
    lab = jnp.pad(label_cls.astype(jnp.int32), ((0, 0), (0, pad)))
    pb = jnp.pad(pred_bboxes, ((0, 0), (0, 0), (0, pad)))
    lt = label_target.reshape(b, 1, 4)

    iou, a1, a2 = pl.pallas_call(
        _iou_prob_body,
        grid=(b,),
        in_specs=[
            pl.BlockSpec((1, 1, NPAD), lambda i: (i, 0, 0)),
            pl.BlockSpec((1, 4, NPAD), lambda i: (i, 0, 0)),
            pl.BlockSpec((1, 1, 4), lambda i: (i, 0, 0)),
        ],
        out_specs=[
            pl.BlockSpec((1, 1, NPAD), lambda i: (i, 0, 0)),
            pl.BlockSpec((1, 1, NPAD), lambda i: (i, 0, 0)),
            pl.BlockSpec((1, 1, NPAD), lambda i: (i, 0, 0)),
        ],
        out_shape=[
            jax.ShapeDtypeStruct((b, 1, NPAD), jnp.float32),
            jax.ShapeDtypeStruct((b, 1, NPAD), jnp.float32),
            jax.ShapeDtypeStruct((b, 1, NPAD), jnp.float32),
        ],
    )(cls1.reshape(b, 1, NPAD), pb, lt)

    iou_r = jnp.swapaxes(iou, 1, 2)
    a1_r = jnp.swapaxes(a1, 1, 2)
    a2_r = jnp.swapaxes(a2, 1, 2)
    cls_c = cls1.reshape(b, 1, NPAD)
    cls_r = cls1.reshape(b, NPAD, 1)
    lab_c = lab.reshape(b, 1, NPAD)
    lab_r = lab.reshape(b, NPAD, 1)

    nb = NPAD // BR
    col = pl.BlockSpec((1, 1, NPAD), lambda i, j: (i, 0, 0))
    row = pl.BlockSpec((1, BR, 1), lambda i, j: (i, j, 0))
    out = pl.BlockSpec((1, 1, NPAD), lambda i, j: (i, 0, 0))

    s1, s2, npos_f = pl.pallas_call(
        _pair_body,
        grid=(b, nb),
        in_specs=[col, col, col, col, col, row, row, row, row, row],
        out_specs=[out, out, out],
        out_shape=[
            jax.ShapeDtypeStruct((b, 1, NPAD), jnp.float32),
            jax.ShapeDtypeStruct((b, 1, NPAD), jnp.float32),
            jax.ShapeDtypeStruct((b, 1, NPAD), jnp.float32),
        ],
    )(iou, a1, a2, cls_c, lab_c, iou_r, a1_r, a2_r, cls_r, lab_r)

    sum1 = s1.sum(axis=(1, 2))
    sum2 = s2.sum(axis=(1, 2))
    num_pos = npos_f.sum(axis=(1, 2)).astype(jnp.int32)

    denom = (num_pos * (num_pos - 1) // 2).astype(jnp.float32)
    loss1 = sum1 / denom
    loss2 = sum2 / denom
    valid = (num_pos > 0) & ~jnp.isnan(loss1) & ~jnp.isnan(loss2)
    cnt = valid.sum().astype(jnp.float32)
    any_valid = valid.any()
    safe_cnt = jnp.where(any_valid, cnt, jnp.asarray(1.0, jnp.float32))
    zero = jnp.asarray(0.0, jnp.float32)
    final1 = jnp.where(any_valid, jnp.where(valid, loss1, 0.0).sum() / safe_cnt, zero)
    final2 = jnp.where(any_valid, jnp.where(valid, loss2, 0.0).sum() / safe_cnt, zero)
    return (final1, final2)


# speedup vs baseline: 6409.8322x; 2.3867x over previous
import functools

import jax
import jax.numpy as jnp
from jax import lax
from jax.experimental import pallas as pl
from jax.experimental.pallas import tpu as pltpu
from jax.experimental.pallas import tpu_sc as plsc

IOU_G = 3.0
LOG2E = 1.4426950408889634
NEGK = -IOU_G * LOG2E
N = 3125
NPAD = 3200
BR = 320
NB = NPAD // BR
NSAMP = 16


def _sc_compact_body(cls_hbm, lab_hbm, pb_hbm, lt_hbm,
                     a1_hbm, a2_hbm, npos_hbm,
                     cls_v, lab_v, pb_v, lt_v, a1_v, a2_v, cnt_v):
    w = lax.axis_index("s") * 2 + lax.axis_index("c")

    @pl.when(w < NSAMP)
    def _():
        pltpu.sync_copy(cls_hbm.at[w], cls_v)
        pltpu.sync_copy(lab_hbm.at[w], lab_v)
        pltpu.sync_copy(pb_hbm.at[w], pb_v)
        pltpu.sync_copy(lt_hbm.at[w], lt_v)

        nan16 = jnp.full((16,), jnp.nan, dtype=jnp.float32)

        def fill(i, carry):
            sl = pl.ds(i * 16, 16)
            a1_v[sl] = nan16
            a2_v[sl] = nan16
            return carry

        lax.fori_loop(0, NPAD // 16, fill, 0)

        tx1 = lt_v[0]
        ty1 = lt_v[1]
        tx2 = lt_v[2]
        ty2 = lt_v[3]
        ta = (tx2 - tx1) * (ty2 - ty1)

        def step(i, base_v):
            sl = pl.ds(i * 16, 16)
            c = cls_v[sl]
            lb = lab_v[sl]
            x1 = pb_v[0, sl]
            y1 = pb_v[1, sl]
            x2 = pb_v[2, sl]
            y2 = pb_v[3, sl]
            ww = jnp.maximum(jnp.minimum(tx2, x2) - jnp.maximum(tx1, x1), 0.0)
            hh = jnp.maximum(jnp.minimum(ty2, y2) - jnp.maximum(ty1, y1), 0.0)
            inter = ww * hh
            iou = inter / ((x2 - x1) * (y2 - y1) + ta - inter)
            a1 = NEGK * jnp.exp(c)
            a2 = NEGK * iou
            m = lb > 0
            mi = m.astype(jnp.int32)
            incl = plsc.cumsum(mi)
            excl = incl - mi
            lanes = lax.broadcasted_iota(jnp.int32, (16,), 0)
            posn = jnp.where(m, base_v + excl, NPAD + lanes)
            plsc.store_scatter(a1_v, [posn], a1)
            plsc.store_scatter(a2_v, [posn], a2)
            return base_v + jnp.broadcast_to(jnp.sum(mi), (16,))

        base0 = jnp.zeros((16,), jnp.int32)
        num_pos_v = lax.fori_loop(0, NPAD // 16, step, base0)

        cnt_v[...] = num_pos_v
        pltpu.sync_copy(a1_v.at[pl.ds(0, NPAD)], a1_hbm.at[w])
        pltpu.sync_copy(a2_v.at[pl.ds(0, NPAD)], a2_hbm.at[w])
        pltpu.sync_copy(cnt_v, npos_hbm.at[w])


_sc_compact = functools.partial(
    pl.kernel,
    _sc_compact_body,
    out_type=(
        jax.ShapeDtypeStruct((NSAMP, NPAD), jnp.float32),
        jax.ShapeDtypeStruct((NSAMP, NPAD), jnp.float32),
        jax.ShapeDtypeStruct((NSAMP, 16), jnp.int32),
    ),
    mesh=plsc.VectorSubcoreMesh(core_axis_name="c", subcore_axis_name="s"),
    compiler_params=pltpu.CompilerParams(needs_layout_passes=False),
    scratch_types=(
        pltpu.VMEM((NPAD,), jnp.float32),
        pltpu.VMEM((NPAD,), jnp.int32),
        pltpu.VMEM((4, NPAD), jnp.float32),
        pltpu.VMEM((4, 16), jnp.float32),
        pltpu.VMEM((NPAD + 16,), jnp.float32),
        pltpu.VMEM((NPAD + 16,), jnp.float32),
        pltpu.VMEM((16,), jnp.int32),
    ),
)


def _pair_body(npos_ref, a1_c_ref, a2_c_ref, a1_r_ref, a2_r_ref,
               s1_ref, s2_ref):
    bidx = pl.program_id(0)
    rb = pl.program_id(1)

    @pl.when(rb == 0)
    def _init():
        s1_ref[0] = jnp.zeros((1, NPAD), jnp.float32)
        s2_ref[0] = jnp.zeros((1, NPAD), jnp.float32)

    @pl.when(rb * BR < npos_ref[bidx])
    def _acc():
        a1_c = a1_c_ref[0]
        a2_c = a2_c_ref[0]
        a1_r = a1_r_ref[0]
        a2_r = a2_r_ref[0]

        idx_c = lax.broadcasted_iota(jnp.int32, (1, NPAD), 1)
        idx_r = lax.broadcasted_iota(jnp.int32, (BR, 1), 0) + rb * BR
        tie = idx_r < idx_c

        o1 = (a2_r < a2_c) | ((a2_r == a2_c) & tie)
        w1 = jnp.exp2(a1_r - a1_c)
        t1 = jnp.where(o1, w1, 0.0)

        o2 = (a1_r < a1_c) | ((a1_r == a1_c) & tie)
        w2 = jnp.exp2(a2_r - a2_c)
        t2 = jnp.where(o2, w2, 0.0)

        ones = jnp.ones((1, BR), jnp.float32)
        part1 = lax.dot_general(ones, t1, (((1,), (0,)), ((), ())),
                                preferred_element_type=jnp.float32)
        part2 = lax.dot_general(ones, t2, (((1,), (0,)), ((), ())),
                                preferred_element_type=jnp.float32)
        s1_ref[0] += part1
        s2_ref[0] += part2


def kernel(cls, label_cls, pred_bboxes, label_target):
    b = label_cls.shape[0]
    pad = NPAD - N

    cls1 = jnp.pad(cls[:, :, 1], ((0, 0), (0, pad)))
    lab = jnp.pad(label_cls.astype(jnp.int32), ((0, 0), (0, pad)))
    pb = jnp.pad(pred_bboxes, ((0, 0), (0, 0), (0, pad)))
    ltb = jnp.broadcast_to(label_target[:, :, None], (b, 4, 16))

    a1, a2, npos_rows = _sc_compact()(cls1, lab, pb, ltb)
    npos = npos_rows[:, 0]

    a1_c = a1.reshape(b, 1, NPAD)
    a2_c = a2.reshape(b, 1, NPAD)
    a1_r = a1.reshape(b, NPAD, 1)
    a2_r = a2.reshape(b, NPAD, 1)

    def _cmap(i, j, n):
        return (i, 0, 0)

    def _rmap(i, j, n):
        return (i, j, 0)

    col = pl.BlockSpec((1, 1, NPAD), _cmap)
    row = pl.BlockSpec((1, BR, 1), _rmap)
    out = pl.BlockSpec((1, 1, NPAD), _cmap)

    grid_spec = pltpu.PrefetchScalarGridSpec(
        num_scalar_prefetch=1,
        grid=(b, NB),
        in_specs=[col, col, row, row],
        out_specs=[out, out],
    )
    s1, s2 = pl.pallas_call(
        _pair_body,
        grid_spec=grid_spec,
        out_shape=[
            jax.ShapeDtypeStruct((b, 1, NPAD), jnp.float32),
            jax.ShapeDtypeStruct((b, 1, NPAD), jnp.float32),
        ],
    )(npos, a1_c, a2_c, a1_r, a2_r)

    sum1 = s1.sum(axis=(1, 2))
    sum2 = s2.sum(axis=(1, 2))

    denom = (npos * (npos - 1) // 2).astype(jnp.float32)
    loss1 = sum1 / denom
    loss2 = sum2 / denom
    valid = (npos > 0) & ~jnp.isnan(loss1) & ~jnp.isnan(loss2)
    cnt = valid.sum().astype(jnp.float32)
    any_valid = valid.any()
    safe_cnt = jnp.where(any_valid, cnt, jnp.asarray(1.0, jnp.float32))
    zero = jnp.asarray(0.0, jnp.float32)
    final1 = jnp.where(any_valid, jnp.where(valid, loss1, 0.0).sum() / safe_cnt, zero)
    final2 = jnp.where(any_valid, jnp.where(valid, loss2, 0.0).sum() / safe_cnt, zero)
    return (final1, final2)


# speedup vs baseline: 7143.0737x; 1.1144x over previous
import functools

import jax
import jax.numpy as jnp
from jax import lax
from jax.experimental import pallas as pl
from jax.experimental.pallas import tpu as pltpu
from jax.experimental.pallas import tpu_sc as plsc

IOU_G = 3.0
LOG2E = 1.4426950408889634
NEGK = -IOU_G * LOG2E
N = 3125
NPAD = 3328
BR = 416
NBR = NPAD // BR
BC = 1664
NBC = NPAD // BC
NSAMP = 16


def _sc_compact_body(cls_hbm, lab_hbm, pb_hbm, lt_hbm,
                     a1_hbm, a2_hbm, npos_hbm,
                     cls_v, lab_v, pb_v, lt_v, a1_v, a2_v, cnt_v):
    w = lax.axis_index("s") * 2 + lax.axis_index("c")

    @pl.when(w < NSAMP)
    def _():
        pltpu.sync_copy(cls_hbm.at[w], cls_v)
        pltpu.sync_copy(lab_hbm.at[w], lab_v)
        pltpu.sync_copy(pb_hbm.at[w], pb_v)
        pltpu.sync_copy(lt_hbm.at[w], lt_v)

        nan16 = jnp.full((16,), jnp.nan, dtype=jnp.float32)

        def fill(i, carry):
            sl = pl.ds(i * 16, 16)
            a1_v[sl] = nan16
            a2_v[sl] = nan16
            return carry

        lax.fori_loop(0, NPAD // 16, fill, 0)

        tx1 = lt_v[0]
        ty1 = lt_v[1]
        tx2 = lt_v[2]
        ty2 = lt_v[3]
        ta = (tx2 - tx1) * (ty2 - ty1)

        def step(i, base_v):
            sl = pl.ds(i * 16, 16)
            c = cls_v[sl]
            lb = lab_v[sl]
            x1 = pb_v[0, sl]
            y1 = pb_v[1, sl]
            x2 = pb_v[2, sl]
            y2 = pb_v[3, sl]
            ww = jnp.maximum(jnp.minimum(tx2, x2) - jnp.maximum(tx1, x1), 0.0)
            hh = jnp.maximum(jnp.minimum(ty2, y2) - jnp.maximum(ty1, y1), 0.0)
            inter = ww * hh
            iou = inter / ((x2 - x1) * (y2 - y1) + ta - inter)
            a1 = NEGK * jnp.exp(c)
            a2 = NEGK * iou
            m = lb > 0
            mi = m.astype(jnp.int32)
            incl = plsc.cumsum(mi)
            excl = incl - mi
            lanes = lax.broadcasted_iota(jnp.int32, (16,), 0)
            posn = jnp.where(m, base_v + excl, NPAD + lanes)
            plsc.store_scatter(a1_v, [posn], a1)
            plsc.store_scatter(a2_v, [posn], a2)
            return base_v + jnp.broadcast_to(jnp.sum(mi), (16,))

        base0 = jnp.zeros((16,), jnp.int32)
        num_pos_v = lax.fori_loop(0, NPAD // 16, step, base0)

        cnt_v[...] = num_pos_v
        pltpu.sync_copy(a1_v.at[pl.ds(0, NPAD)], a1_hbm.at[w])
        pltpu.sync_copy(a2_v.at[pl.ds(0, NPAD)], a2_hbm.at[w])
        pltpu.sync_copy(cnt_v, npos_hbm.at[w])


_sc_compact = functools.partial(
    pl.kernel,
    _sc_compact_body,
    out_type=(
        jax.ShapeDtypeStruct((NSAMP, NPAD), jnp.float32),
        jax.ShapeDtypeStruct((NSAMP, NPAD), jnp.float32),
        jax.ShapeDtypeStruct((NSAMP, 16), jnp.int32),
    ),
    mesh=plsc.VectorSubcoreMesh(core_axis_name="c", subcore_axis_name="s"),
    compiler_params=pltpu.CompilerParams(needs_layout_passes=False),
    scratch_types=(
        pltpu.VMEM((NPAD,), jnp.float32),
        pltpu.VMEM((NPAD,), jnp.int32),
        pltpu.VMEM((4, NPAD), jnp.float32),
        pltpu.VMEM((4, 16), jnp.float32),
        pltpu.VMEM((NPAD + 16,), jnp.float32),
        pltpu.VMEM((NPAD + 16,), jnp.float32),
        pltpu.VMEM((16,), jnp.int32),
    ),
)


def _pair_body(npos_ref, a1_c_ref, a2_c_ref, a1_r_ref, a2_r_ref,
               s1_ref, s2_ref):
    bidx = pl.program_id(0)
    cb = pl.program_id(1)
    rb = pl.program_id(2)

    @pl.when(rb == 0)
    def _init():
        s1_ref[0] = jnp.zeros((1, BC), jnp.float32)
        s2_ref[0] = jnp.zeros((1, BC), jnp.float32)

    npos = npos_ref[bidx]

    @pl.when((rb * BR < npos) & (cb * BC < npos))
    def _acc():
        a1_c = a1_c_ref[0]
        a2_c = a2_c_ref[0]
        a1_r = a1_r_ref[0]
        a2_r = a2_r_ref[0]

        idx_c = lax.broadcasted_iota(jnp.int32, (1, BC), 1) + cb * BC
        idx_r = lax.broadcasted_iota(jnp.int32, (BR, 1), 0) + rb * BR
        tie = idx_r < idx_c

        o1 = (a2_r < a2_c) | ((a2_r == a2_c) & tie)
        w1 = jnp.exp2(a1_r - a1_c)
        t1 = jnp.where(o1, w1, 0.0)

        o2 = (a1_r < a1_c) | ((a1_r == a1_c) & tie)
        w2 = jnp.exp2(a2_r - a2_c)
        t2 = jnp.where(o2, w2, 0.0)

        ones = jnp.ones((1, BR), jnp.float32)
        part1 = lax.dot_general(ones, t1, (((1,), (0,)), ((), ())),
                                preferred_element_type=jnp.float32)
        part2 = lax.dot_general(ones, t2, (((1,), (0,)), ((), ())),
                                preferred_element_type=jnp.float32)
        s1_ref[0] += part1
        s2_ref[0] += part2


def kernel(cls, label_cls, pred_bboxes, label_target):
    b = label_cls.shape[0]
    pad = NPAD - N

    cls1 = jnp.pad(cls[:, :, 1], ((0, 0), (0, pad)))
    lab = jnp.pad(label_cls.astype(jnp.int32), ((0, 0), (0, pad)))
    pb = jnp.pad(pred_bboxes, ((0, 0), (0, 0), (0, pad)))
    ltb = jnp.broadcast_to(label_target[:, :, None], (b, 4, 16))

    a1, a2, npos_rows = _sc_compact()(cls1, lab, pb, ltb)
    npos = npos_rows[:, 0]

    a1_c = a1.reshape(b, 1, NPAD)
    a2_c = a2.reshape(b, 1, NPAD)
    a1_r = a1.reshape(b, NPAD, 1)
    a2_r = a2.reshape(b, NPAD, 1)

    def _cmap(i, cb, rb, n):
        return (i, 0, cb)

    def _rmap(i, cb, rb, n):
        return (i, rb, 0)

    col = pl.BlockSpec((1, 1, BC), _cmap)
    row = pl.BlockSpec((1, BR, 1), _rmap)
    out = pl.BlockSpec((1, 1, BC), _cmap)

    grid_spec = pltpu.PrefetchScalarGridSpec(
        num_scalar_prefetch=1,
        grid=(b, NBC, NBR),
        in_specs=[col, col, row, row],
        out_specs=[out, out],
    )
    s1, s2 = pl.pallas_call(
        _pair_body,
        grid_spec=grid_spec,
        out_shape=[
            jax.ShapeDtypeStruct((b, 1, NPAD), jnp.float32),
            jax.ShapeDtypeStruct((b, 1, NPAD), jnp.float32),
        ],
    )(npos, a1_c, a2_c, a1_r, a2_r)

    sum1 = s1.sum(axis=(1, 2))
    sum2 = s2.sum(axis=(1, 2))

    denom = (npos * (npos - 1) // 2).astype(jnp.float32)
    loss1 = sum1 / denom
    loss2 = sum2 / denom
    valid = (npos > 0) & ~jnp.isnan(loss1) & ~jnp.isnan(loss2)
    cnt = valid.sum().astype(jnp.float32)
    any_valid = valid.any()
    safe_cnt = jnp.where(any_valid, cnt, jnp.asarray(1.0, jnp.float32))
    zero = jnp.asarray(0.0, jnp.float32)
    final1 = jnp.where(any_valid, jnp.where(valid, loss1, 0.0).sum() / safe_cnt, zero)
    final2 = jnp.where(any_valid, jnp.where(valid, loss2, 0.0).sum() / safe_cnt, zero)
    return (final1, final2)


# speedup vs baseline: 8466.7829x; 1.1853x over previous
import functools

import jax
import jax.numpy as jnp
from jax import lax
from jax.experimental import pallas as pl
from jax.experimental.pallas import tpu as pltpu
from jax.experimental.pallas import tpu_sc as plsc

IOU_G = 3.0
LOG2E = 1.4426950408889634
NEGK = -IOU_G * LOG2E
N = 3125
NPAD = 3328
BR = 832
NBR = NPAD // BR
BC = 1664
NBC = NPAD // BC
NSAMP = 16


def _sc_compact_body(cls_hbm, lab_hbm, pb_hbm, lt_hbm,
                     a1_hbm, a2_hbm, npos_hbm,
                     cls_v, lab_v, pb_v, lt_v, a1_v, a2_v, cnt_v):
    w = lax.axis_index("s") * 2 + lax.axis_index("c")

    @pl.when(w < NSAMP)
    def _():
        pltpu.sync_copy(cls_hbm.at[w], cls_v)
        pltpu.sync_copy(lab_hbm.at[w], lab_v)
        pltpu.sync_copy(pb_hbm.at[w], pb_v)
        pltpu.sync_copy(lt_hbm.at[w], lt_v)

        nan16 = jnp.full((16,), jnp.nan, dtype=jnp.float32)

        def fill(i, carry):
            sl = pl.ds(i * 16, 16)
            a1_v[sl] = nan16
            a2_v[sl] = nan16
            return carry

        lax.fori_loop(0, NPAD // 16, fill, 0)

        tx1 = lt_v[0]
        ty1 = lt_v[1]
        tx2 = lt_v[2]
        ty2 = lt_v[3]
        ta = (tx2 - tx1) * (ty2 - ty1)

        def step(i, base_v):
            sl = pl.ds(i * 16, 16)
            c = cls_v[sl]
            lb = lab_v[sl]
            x1 = pb_v[0, sl]
            y1 = pb_v[1, sl]
            x2 = pb_v[2, sl]
            y2 = pb_v[3, sl]
            ww = jnp.maximum(jnp.minimum(tx2, x2) - jnp.maximum(tx1, x1), 0.0)
            hh = jnp.maximum(jnp.minimum(ty2, y2) - jnp.maximum(ty1, y1), 0.0)
            inter = ww * hh
            iou = inter / ((x2 - x1) * (y2 - y1) + ta - inter)
            a1 = NEGK * jnp.exp(c)
            a2 = NEGK * iou
            m = lb > 0
            mi = m.astype(jnp.int32)
            incl = plsc.cumsum(mi)
            excl = incl - mi
            lanes = lax.broadcasted_iota(jnp.int32, (16,), 0)
            posn = jnp.where(m, base_v + excl, NPAD + lanes)
            plsc.store_scatter(a1_v, [posn], a1)
            plsc.store_scatter(a2_v, [posn], a2)
            return base_v + jnp.broadcast_to(jnp.sum(mi), (16,))

        base0 = jnp.zeros((16,), jnp.int32)
        num_pos_v = lax.fori_loop(0, NPAD // 16, step, base0)

        cnt_v[...] = num_pos_v
        pltpu.sync_copy(a1_v.at[pl.ds(0, NPAD)], a1_hbm.at[w])
        pltpu.sync_copy(a2_v.at[pl.ds(0, NPAD)], a2_hbm.at[w])
        pltpu.sync_copy(cnt_v, npos_hbm.at[w])


_sc_compact = functools.partial(
    pl.kernel,
    _sc_compact_body,
    out_type=(
        jax.ShapeDtypeStruct((NSAMP, NPAD), jnp.float32),
        jax.ShapeDtypeStruct((NSAMP, NPAD), jnp.float32),
        jax.ShapeDtypeStruct((NSAMP, 16), jnp.int32),
    ),
    mesh=plsc.VectorSubcoreMesh(core_axis_name="c", subcore_axis_name="s"),
    compiler_params=pltpu.CompilerParams(needs_layout_passes=False),
    scratch_types=(
        pltpu.VMEM((NPAD,), jnp.float32),
        pltpu.VMEM((NPAD,), jnp.int32),
        pltpu.VMEM((4, NPAD), jnp.float32),
        pltpu.VMEM((4, 16), jnp.float32),
        pltpu.VMEM((NPAD + 16,), jnp.float32),
        pltpu.VMEM((NPAD + 16,), jnp.float32),
        pltpu.VMEM((16,), jnp.int32),
    ),
)


def _pair_body(npos_ref, a1_c_ref, a2_c_ref, a1_r_ref, a2_r_ref,
               s1_ref, s2_ref):
    bidx = pl.program_id(0)
    cb = pl.program_id(1)
    rb = pl.program_id(2)

    @pl.when(rb == 0)
    def _init():
        s1_ref[0] = jnp.zeros((1, BC), jnp.float32)
        s2_ref[0] = jnp.zeros((1, BC), jnp.float32)

    npos = npos_ref[bidx]

    @pl.when((rb * BR < npos) & (cb * BC < npos))
    def _acc():
        a1_c = a1_c_ref[0]
        a2_c = a2_c_ref[0]
        a1_r = a1_r_ref[0]
        a2_r = a2_r_ref[0]

        idx_c = lax.broadcasted_iota(jnp.int32, (1, BC), 1) + cb * BC
        idx_r = lax.broadcasted_iota(jnp.int32, (BR, 1), 0) + rb * BR
        tie = idx_r < idx_c

        o1 = (a2_r < a2_c) | ((a2_r == a2_c) & tie)
        w1 = jnp.exp2(a1_r - a1_c)
        t1 = jnp.where(o1, w1, 0.0)

        o2 = (a1_r < a1_c) | ((a1_r == a1_c) & tie)
        w2 = jnp.exp2(a2_r - a2_c)
        t2 = jnp.where(o2, w2, 0.0)

        ones = jnp.ones((1, BR), jnp.float32)
        part1 = lax.dot_general(ones, t1, (((1,), (0,)), ((), ())),
                                preferred_element_type=jnp.float32)
        part2 = lax.dot_general(ones, t2, (((1,), (0,)), ((), ())),
                                preferred_element_type=jnp.float32)
        s1_ref[0] += part1
        s2_ref[0] += part2


def kernel(cls, label_cls, pred_bboxes, label_target):
    b = label_cls.shape[0]
    pad = NPAD - N

    cls1 = jnp.pad(cls[:, :, 1], ((0, 0), (0, pad)))
    lab = jnp.pad(label_cls.astype(jnp.int32), ((0, 0), (0, pad)))
    pb = jnp.pad(pred_bboxes, ((0, 0), (0, 0), (0, pad)))
    ltb = jnp.broadcast_to(label_target[:, :, None], (b, 4, 16))

    a1, a2, npos_rows = _sc_compact()(cls1, lab, pb, ltb)
    npos = npos_rows[:, 0]

    a1_c = a1.reshape(b, 1, NPAD)
    a2_c = a2.reshape(b, 1, NPAD)
    a1_r = a1.reshape(b, NPAD, 1)
    a2_r = a2.reshape(b, NPAD, 1)

    def _cmap(i, cb, rb, n):
        return (i, 0, cb)

    def _rmap(i, cb, rb, n):
        return (i, rb, 0)

    col = pl.BlockSpec((1, 1, BC), _cmap)
    row = pl.BlockSpec((1, BR, 1), _rmap)
    out = pl.BlockSpec((1, 1, BC), _cmap)

    grid_spec = pltpu.PrefetchScalarGridSpec(
        num_scalar_prefetch=1,
        grid=(b, NBC, NBR),
        in_specs=[col, col, row, row],
        out_specs=[out, out],
    )
    s1, s2 = pl.pallas_call(
        _pair_body,
        grid_spec=grid_spec,
        out_shape=[
            jax.ShapeDtypeStruct((b, 1, NPAD), jnp.float32),
            jax.ShapeDtypeStruct((b, 1, NPAD), jnp.float32),
        ],
    )(npos, a1_c, a2_c, a1_r, a2_r)

    sum1 = s1.sum(axis=(1, 2))
    sum2 = s2.sum(axis=(1, 2))

    denom = (npos * (npos - 1) // 2).astype(jnp.float32)
    loss1 = sum1 / denom
    loss2 = sum2 / denom
    valid = (npos > 0) & ~jnp.isnan(loss1) & ~jnp.isnan(loss2)
    cnt = valid.sum().astype(jnp.float32)
    any_valid = valid.any()
    safe_cnt = jnp.where(any_valid, cnt, jnp.asarray(1.0, jnp.float32))
    zero = jnp.asarray(0.0, jnp.float32)
    final1 = jnp.where(any_valid, jnp.where(valid, loss1, 0.0).sum() / safe_cnt, zero)
    final2 = jnp.where(any_valid, jnp.where(valid, loss2, 0.0).sum() / safe_cnt, zero)
    return (final1, final2)


# speedup vs baseline: 9198.2456x; 1.0864x over previous
import functools

import jax
import jax.numpy as jnp
from jax import lax
from jax.experimental import pallas as pl
from jax.experimental.pallas import tpu as pltpu
from jax.experimental.pallas import tpu_sc as plsc

IOU_G = 3.0
LOG2E = 1.4426950408889634
NEGK = -IOU_G * LOG2E
N = 3125
NPAD = 3328
BR = 1664
NBR = NPAD // BR
BC = 1664
NBC = NPAD // BC
NSAMP = 16


def _sc_compact_body(cls_hbm, lab_hbm, pb_hbm, lt_hbm,
                     a1_hbm, a2_hbm, npos_hbm,
                     cls_v, lab_v, pb_v, lt_v, a1_v, a2_v, cnt_v):
    w = lax.axis_index("s") * 2 + lax.axis_index("c")

    @pl.when(w < NSAMP)
    def _():
        pltpu.sync_copy(cls_hbm.at[w], cls_v)
        pltpu.sync_copy(lab_hbm.at[w], lab_v)
        pltpu.sync_copy(pb_hbm.at[w], pb_v)
        pltpu.sync_copy(lt_hbm.at[w], lt_v)

        nan16 = jnp.full((16,), jnp.nan, dtype=jnp.float32)

        def fill(i, carry):
            sl = pl.ds(i * 16, 16)
            a1_v[sl] = nan16
            a2_v[sl] = nan16
            return carry

        lax.fori_loop(0, NPAD // 16, fill, 0)

        tx1 = lt_v[0]
        ty1 = lt_v[1]
        tx2 = lt_v[2]
        ty2 = lt_v[3]
        ta = (tx2 - tx1) * (ty2 - ty1)

        def step(i, base_v):
            sl = pl.ds(i * 16, 16)
            c = cls_v[sl]
            lb = lab_v[sl]
            x1 = pb_v[0, sl]
            y1 = pb_v[1, sl]
            x2 = pb_v[2, sl]
            y2 = pb_v[3, sl]
            ww = jnp.maximum(jnp.minimum(tx2, x2) - jnp.maximum(tx1, x1), 0.0)
            hh = jnp.maximum(jnp.minimum(ty2, y2) - jnp.maximum(ty1, y1), 0.0)
            inter = ww * hh
            iou = inter / ((x2 - x1) * (y2 - y1) + ta - inter)
            a1 = NEGK * jnp.exp(c)
            a2 = NEGK * iou
            m = lb > 0
            mi = m.astype(jnp.int32)
            incl = plsc.cumsum(mi)
            excl = incl - mi
            lanes = lax.broadcasted_iota(jnp.int32, (16,), 0)
            posn = jnp.where(m, base_v + excl, NPAD + lanes)
            plsc.store_scatter(a1_v, [posn], a1)
            plsc.store_scatter(a2_v, [posn], a2)
            return base_v + jnp.broadcast_to(jnp.sum(mi), (16,))

        base0 = jnp.zeros((16,), jnp.int32)
        num_pos_v = lax.fori_loop(0, NPAD // 16, step, base0)

        cnt_v[...] = num_pos_v
        pltpu.sync_copy(a1_v.at[pl.ds(0, NPAD)], a1_hbm.at[w])
        pltpu.sync_copy(a2_v.at[pl.ds(0, NPAD)], a2_hbm.at[w])
        pltpu.sync_copy(cnt_v, npos_hbm.at[w])


_sc_compact = functools.partial(
    pl.kernel,
    _sc_compact_body,
    out_type=(
        jax.ShapeDtypeStruct((NSAMP, NPAD), jnp.float32),
        jax.ShapeDtypeStruct((NSAMP, NPAD), jnp.float32),
        jax.ShapeDtypeStruct((NSAMP, 16), jnp.int32),
    ),
    mesh=plsc.VectorSubcoreMesh(core_axis_name="c", subcore_axis_name="s"),
    compiler_params=pltpu.CompilerParams(needs_layout_passes=False),
    scratch_types=(
        pltpu.VMEM((NPAD,), jnp.float32),
        pltpu.VMEM((NPAD,), jnp.int32),
        pltpu.VMEM((4, NPAD), jnp.float32),
        pltpu.VMEM((4, 16), jnp.float32),
        pltpu.VMEM((NPAD + 16,), jnp.float32),
        pltpu.VMEM((NPAD + 16,), jnp.float32),
        pltpu.VMEM((16,), jnp.int32),
    ),
)


def _pair_body(npos_ref, a1_c_ref, a2_c_ref, a1_r_ref, a2_r_ref,
               s1_ref, s2_ref):
    bidx = pl.program_id(0)
    cb = pl.program_id(1)
    rb = pl.program_id(2)

    @pl.when(rb == 0)
    def _init():
        s1_ref[0] = jnp.zeros((1, BC), jnp.float32)
        s2_ref[0] = jnp.zeros((1, BC), jnp.float32)

    npos = npos_ref[bidx]

    @pl.when((rb * BR < npos) & (cb * BC < npos))
    def _acc():
        a1_c = a1_c_ref[0]
        a2_c = a2_c_ref[0]
        a1_r = a1_r_ref[0]
        a2_r = a2_r_ref[0]

        idx_c = lax.broadcasted_iota(jnp.int32, (1, BC), 1) + cb * BC
        idx_r = lax.broadcasted_iota(jnp.int32, (BR, 1), 0) + rb * BR
        tie = idx_r < idx_c

        o1 = (a2_r < a2_c) | ((a2_r == a2_c) & tie)
        w1 = jnp.exp2(a1_r - a1_c)
        t1 = jnp.where(o1, w1, 0.0)

        o2 = (a1_r < a1_c) | ((a1_r == a1_c) & tie)
        w2 = jnp.exp2(a2_r - a2_c)
        t2 = jnp.where(o2, w2, 0.0)

        ones = jnp.ones((1, BR), jnp.float32)
        part1 = lax.dot_general(ones, t1, (((1,), (0,)), ((), ())),
                                preferred_element_type=jnp.float32)
        part2 = lax.dot_general(ones, t2, (((1,), (0,)), ((), ())),
                                preferred_element_type=jnp.float32)
        s1_ref[0] += part1
        s2_ref[0] += part2


def kernel(cls, label_cls, pred_bboxes, label_target):
    b = label_cls.shape[0]
    pad = NPAD - N

    cls1 = jnp.pad(cls[:, :, 1], ((0, 0), (0, pad)))
    lab = jnp.pad(label_cls.astype(jnp.int32), ((0, 0), (0, pad)))
    pb = jnp.pad(pred_bboxes, ((0, 0), (0, 0), (0, pad)))
    ltb = jnp.broadcast_to(label_target[:, :, None], (b, 4, 16))

    a1, a2, npos_rows = _sc_compact()(cls1, lab, pb, ltb)
    npos = npos_rows[:, 0]

    a1_c = a1.reshape(b, 1, NPAD)
    a2_c = a2.reshape(b, 1, NPAD)
    a1_r = a1.reshape(b, NPAD, 1)
    a2_r = a2.reshape(b, NPAD, 1)

    def _cmap(i, cb, rb, n):
        return (i, 0, cb)

    def _rmap(i, cb, rb, n):
        return (i, rb, 0)

    col = pl.BlockSpec((1, 1, BC), _cmap)
    row = pl.BlockSpec((1, BR, 1), _rmap)
    out = pl.BlockSpec((1, 1, BC), _cmap)

    grid_spec = pltpu.PrefetchScalarGridSpec(
        num_scalar_prefetch=1,
        grid=(b, NBC, NBR),
        in_specs=[col, col, row, row],
        out_specs=[out, out],
    )
    s1, s2 = pl.pallas_call(
        _pair_body,
        grid_spec=grid_spec,
        out_shape=[
            jax.ShapeDtypeStruct((b, 1, NPAD), jnp.float32),
            jax.ShapeDtypeStruct((b, 1, NPAD), jnp.float32),
        ],
    )(npos, a1_c, a2_c, a1_r, a2_r)

    sum1 = s1.sum(axis=(1, 2))
    sum2 = s2.sum(axis=(1, 2))

    denom = (npos * (npos - 1) // 2).astype(jnp.float32)
    loss1 = sum1 / denom
    loss2 = sum2 / denom
    valid = (npos > 0) & ~jnp.isnan(loss1) & ~jnp.isnan(loss2)
    cnt = valid.sum().astype(jnp.float32)
    any_valid = valid.any()
    safe_cnt = jnp.where(any_valid, cnt, jnp.asarray(1.0, jnp.float32))
    zero = jnp.asarray(0.0, jnp.float32)
    final1 = jnp.where(any_valid, jnp.where(valid, loss1, 0.0).sum() / safe_cnt, zero)
    final2 = jnp.where(any_valid, jnp.where(valid, loss2, 0.0).sum() / safe_cnt, zero)
    return (final1, final2)


# speedup vs baseline: 11458.8027x; 1.2458x over previous
import functools

import jax
import jax.numpy as jnp
from jax import lax
from jax.experimental import pallas as pl
from jax.experimental.pallas import tpu as pltpu
from jax.experimental.pallas import tpu_sc as plsc

IOU_G = 3.0
LOG2E = 1.4426950408889634
NEGK = -IOU_G * LOG2E
N = 3125
NPAD = 3328
BR = 1664
NBR = NPAD // BR
BC = 1664
NBC = NPAD // BC
NSAMP = 16


def _sc_compact_body(cls_hbm, lab_hbm, pb_hbm, lt_hbm,
                     a1_hbm, a2_hbm, npos_hbm,
                     cls_v, lab_v, pb_v, lt_v, a1_v, a2_v, cnt_v):
    w = lax.axis_index("s") * 2 + lax.axis_index("c")

    @pl.when(w < NSAMP)
    def _():
        pltpu.sync_copy(cls_hbm.at[w], cls_v)
        pltpu.sync_copy(lab_hbm.at[w], lab_v)
        pltpu.sync_copy(pb_hbm.at[w], pb_v)
        pltpu.sync_copy(lt_hbm.at[w], lt_v)

        nan16 = jnp.full((16,), jnp.nan, dtype=jnp.float32)

        def fill(i, carry):
            sl = pl.ds(i * 16, 16)
            a1_v[sl] = nan16
            a2_v[sl] = nan16
            return carry

        lax.fori_loop(0, NPAD // 16, fill, 0)

        tx1 = lt_v[0]
        ty1 = lt_v[1]
        tx2 = lt_v[2]
        ty2 = lt_v[3]
        ta = (tx2 - tx1) * (ty2 - ty1)

        def step(i, base_v):
            sl = pl.ds(i * 16, 16)
            c = cls_v[sl]
            lb = lab_v[sl]
            x1 = pb_v[0, sl]
            y1 = pb_v[1, sl]
            x2 = pb_v[2, sl]
            y2 = pb_v[3, sl]
            ww = jnp.maximum(jnp.minimum(tx2, x2) - jnp.maximum(tx1, x1), 0.0)
            hh = jnp.maximum(jnp.minimum(ty2, y2) - jnp.maximum(ty1, y1), 0.0)
            inter = ww * hh
            iou = inter / ((x2 - x1) * (y2 - y1) + ta - inter)
            a1 = NEGK * jnp.exp(c)
            a2 = NEGK * iou
            m = lb > 0
            mi = m.astype(jnp.int32)
            incl = plsc.cumsum(mi)
            excl = incl - mi
            lanes = lax.broadcasted_iota(jnp.int32, (16,), 0)
            posn = jnp.where(m, base_v + excl, NPAD + lanes)
            plsc.store_scatter(a1_v, [posn], a1)
            plsc.store_scatter(a2_v, [posn], a2)
            return base_v + jnp.broadcast_to(jnp.sum(mi), (16,))

        base0 = jnp.zeros((16,), jnp.int32)
        num_pos_v = lax.fori_loop(0, NPAD // 16, step, base0)

        cnt_v[...] = num_pos_v
        pltpu.sync_copy(a1_v.at[pl.ds(0, NPAD)], a1_hbm.at[w])
        pltpu.sync_copy(a2_v.at[pl.ds(0, NPAD)], a2_hbm.at[w])
        pltpu.sync_copy(cnt_v, npos_hbm.at[w])


_sc_compact = functools.partial(
    pl.kernel,
    _sc_compact_body,
    out_type=(
        jax.ShapeDtypeStruct((NSAMP, NPAD), jnp.float32),
        jax.ShapeDtypeStruct((NSAMP, NPAD), jnp.float32),
        jax.ShapeDtypeStruct((NSAMP, 16), jnp.int32),
    ),
    mesh=plsc.VectorSubcoreMesh(core_axis_name="c", subcore_axis_name="s"),
    compiler_params=pltpu.CompilerParams(needs_layout_passes=False),
    scratch_types=(
        pltpu.VMEM((NPAD,), jnp.float32),
        pltpu.VMEM((NPAD,), jnp.int32),
        pltpu.VMEM((4, NPAD), jnp.float32),
        pltpu.VMEM((4, 16), jnp.float32),
        pltpu.VMEM((NPAD + 16,), jnp.float32),
        pltpu.VMEM((NPAD + 16,), jnp.float32),
        pltpu.VMEM((16,), jnp.int32),
    ),
)


def _pair_body(npos_ref, a1_c_ref, a2_c_ref, a1_r_ref, a2_r_ref,
               s1_ref, s2_ref):
    bidx = pl.program_id(0)
    cb = pl.program_id(1)
    rb = pl.program_id(2)

    @pl.when(rb == 0)
    def _init():
        s1_ref[0] = jnp.zeros((1, BC), jnp.float32)
        s2_ref[0] = jnp.zeros((1, BC), jnp.float32)

    npos = npos_ref[bidx]

    @pl.when((rb * BR < npos) & (cb * BC < npos))
    def _acc():
        a1_c = a1_c_ref[0]
        a2_c = a2_c_ref[0]
        a1_r = jnp.transpose(a1_r_ref[0], (1, 0))
        a2_r = jnp.transpose(a2_r_ref[0], (1, 0))

        idx_c = lax.broadcasted_iota(jnp.int32, (1, BC), 1) + cb * BC
        idx_r = lax.broadcasted_iota(jnp.int32, (BR, 1), 0) + rb * BR
        tie = idx_r < idx_c

        o1 = (a2_r < a2_c) | ((a2_r == a2_c) & tie)
        w1 = jnp.exp2(a1_r - a1_c)
        t1 = jnp.where(o1, w1, 0.0)

        o2 = (a1_r < a1_c) | ((a1_r == a1_c) & tie)
        w2 = jnp.exp2(a2_r - a2_c)
        t2 = jnp.where(o2, w2, 0.0)

        ones = jnp.ones((1, BR), jnp.float32)
        part1 = lax.dot_general(ones, t1, (((1,), (0,)), ((), ())),
                                preferred_element_type=jnp.float32)
        part2 = lax.dot_general(ones, t2, (((1,), (0,)), ((), ())),
                                preferred_element_type=jnp.float32)
        s1_ref[0] += part1
        s2_ref[0] += part2


def kernel(cls, label_cls, pred_bboxes, label_target):
    b = label_cls.shape[0]
    pad = NPAD - N

    cls1 = jnp.pad(cls[:, :, 1], ((0, 0), (0, pad)))
    lab = jnp.pad(label_cls.astype(jnp.int32), ((0, 0), (0, pad)))
    pb = jnp.pad(pred_bboxes, ((0, 0), (0, 0), (0, pad)))
    ltb = jnp.broadcast_to(label_target[:, :, None], (b, 4, 16))

    a1, a2, npos_rows = _sc_compact()(cls1, lab, pb, ltb)
    npos = npos_rows[:, 0]

    a1_c = a1.reshape(b, 1, NPAD)
    a2_c = a2.reshape(b, 1, NPAD)

    def _cmap(i, cb, rb, n):
        return (i, 0, cb)

    def _rmap(i, cb, rb, n):
        return (i, 0, rb)

    col = pl.BlockSpec((1, 1, BC), _cmap)
    row = pl.BlockSpec((1, 1, BR), _rmap)
    out = pl.BlockSpec((1, 1, BC), _cmap)

    grid_spec = pltpu.PrefetchScalarGridSpec(
        num_scalar_prefetch=1,
        grid=(b, NBC, NBR),
        in_specs=[col, col, row, row],
        out_specs=[out, out],
    )
    s1, s2 = pl.pallas_call(
        _pair_body,
        grid_spec=grid_spec,
        out_shape=[
            jax.ShapeDtypeStruct((b, 1, NPAD), jnp.float32),
            jax.ShapeDtypeStruct((b, 1, NPAD), jnp.float32),
        ],
    )(npos, a1_c, a2_c, a1_c, a2_c)

    sum1 = s1.sum(axis=(1, 2))
    sum2 = s2.sum(axis=(1, 2))

    denom = (npos * (npos - 1) // 2).astype(jnp.float32)
    loss1 = sum1 / denom
    loss2 = sum2 / denom
    valid = (npos > 0) & ~jnp.isnan(loss1) & ~jnp.isnan(loss2)
    cnt = valid.sum().astype(jnp.float32)
    any_valid = valid.any()
    safe_cnt = jnp.where(any_valid, cnt, jnp.asarray(1.0, jnp.float32))
    zero = jnp.asarray(0.0, jnp.float32)
    final1 = jnp.where(any_valid, jnp.where(valid, loss1, 0.0).sum() / safe_cnt, zero)
    final2 = jnp.where(any_valid, jnp.where(valid, loss2, 0.0).sum() / safe_cnt, zero)
    return (final1, final2)


# speedup vs baseline: 11462.1977x; 1.0003x over previous
import jax
import jax.numpy as jnp
from jax import lax
from jax.experimental import pallas as pl
from jax.experimental.pallas import tpu as pltpu
from jax.experimental.pallas import tpu_sc as plsc

IOU_G = 3.0
LOG2E = 1.4426950408889634
NEGK = -IOU_G * LOG2E
N = 3125
NPAD = 3328
BR = 1664
NBR = NPAD // BR
BC = 1664
NBC = NPAD // BC
NSAMP = 16


def _sc_compact_body(cls_hbm, lab_hbm, pb_hbm, lt_hbm,
                     a1_hbm, a2_hbm, npos_hbm,
                     cls_v, lab_v, pb_v, lt_v, a1_v, a2_v, cnt_v):
    w = lax.axis_index("s") * 2 + lax.axis_index("c")

    @pl.when(w < NSAMP)
    def _():
        pltpu.sync_copy(cls_hbm.at[w], cls_v)
        pltpu.sync_copy(lab_hbm.at[w], lab_v)
        pltpu.sync_copy(pb_hbm.at[w], pb_v)
        pltpu.sync_copy(lt_hbm.at[w], lt_v)

        nan16 = jnp.full((16,), jnp.nan, dtype=jnp.float32)

        def fill(i, carry):
            sl = pl.ds(i * 16, 16)
            a1_v[sl] = nan16
            a2_v[sl] = nan16
            return carry

        lax.fori_loop(0, NPAD // 16, fill, 0)

        tx1 = lt_v[0]
        ty1 = lt_v[1]
        tx2 = lt_v[2]
        ty2 = lt_v[3]
        ta = (tx2 - tx1) * (ty2 - ty1)

        def step(i, base_v):
            sl = pl.ds(i * 16, 16)
            c = cls_v[sl]
            lb = lab_v[sl]
            x1 = pb_v[0, sl]
            y1 = pb_v[1, sl]
            x2 = pb_v[2, sl]
            y2 = pb_v[3, sl]
            ww = jnp.maximum(jnp.minimum(tx2, x2) - jnp.maximum(tx1, x1), 0.0)
            hh = jnp.maximum(jnp.minimum(ty2, y2) - jnp.maximum(ty1, y1), 0.0)
            inter = ww * hh
            iou = inter / ((x2 - x1) * (y2 - y1) + ta - inter)
            a1 = NEGK * jnp.exp(c)
            a2 = NEGK * iou
            m = lb > 0
            mi = m.astype(jnp.int32)
            incl = plsc.cumsum(mi)
            excl = incl - mi
            lanes = lax.broadcasted_iota(jnp.int32, (16,), 0)
            posn = jnp.where(m, base_v + excl, NPAD + lanes)
            plsc.store_scatter(a1_v, [posn], a1)
            plsc.store_scatter(a2_v, [posn], a2)
            return base_v + jnp.broadcast_to(jnp.sum(mi), (16,))

        base0 = jnp.zeros((16,), jnp.int32)
        num_pos_v = lax.fori_loop(0, NPAD // 16, step, base0)

        cnt_v[...] = num_pos_v
        pltpu.sync_copy(a1_v.at[pl.ds(0, NPAD)], a1_hbm.at[w])
        pltpu.sync_copy(a2_v.at[pl.ds(0, NPAD)], a2_hbm.at[w])
        pltpu.sync_copy(cnt_v, npos_hbm.at[w])


def _sc_compact():
    return pl.kernel(
        _sc_compact_body,
        out_type=(
            jax.ShapeDtypeStruct((NSAMP, NPAD), jnp.float32),
            jax.ShapeDtypeStruct((NSAMP, NPAD), jnp.float32),
            jax.ShapeDtypeStruct((NSAMP, 16), jnp.int32),
        ),
        mesh=plsc.VectorSubcoreMesh(core_axis_name="c", subcore_axis_name="s"),
        compiler_params=pltpu.CompilerParams(needs_layout_passes=False),
        scratch_types=(
            pltpu.VMEM((NPAD,), jnp.float32),
            pltpu.VMEM((NPAD,), jnp.int32),
            pltpu.VMEM((4, NPAD), jnp.float32),
            pltpu.VMEM((4, 16), jnp.float32),
            pltpu.VMEM((NPAD + 16,), jnp.float32),
            pltpu.VMEM((NPAD + 16,), jnp.float32),
            pltpu.VMEM((16,), jnp.int32),
        ),
    )


def _pair_body(npos_ref, a1_c_ref, a2_c_ref, a1_r_ref, a2_r_ref,
               s1_ref, s2_ref):
    bidx = pl.program_id(0)
    cb = pl.program_id(1)
    rb = pl.program_id(2)

    @pl.when(rb == 0)
    def _init():
        s1_ref[0] = jnp.zeros((1, BC), jnp.float32)
        s2_ref[0] = jnp.zeros((1, BC), jnp.float32)

    npos = npos_ref[bidx]

    @pl.when((rb * BR < npos) & (cb * BC < npos))
    def _acc():
        a1_c = a1_c_ref[0]
        a2_c = a2_c_ref[0]
        a1_r = jnp.transpose(a1_r_ref[0], (1, 0))
        a2_r = jnp.transpose(a2_r_ref[0], (1, 0))

        idx_c = lax.broadcasted_iota(jnp.int32, (1, BC), 1) + cb * BC
        idx_r = lax.broadcasted_iota(jnp.int32, (BR, 1), 0) + rb * BR
        tie = idx_r < idx_c

        o1 = (a2_r < a2_c) | ((a2_r == a2_c) & tie)
        w1 = jnp.exp2(a1_r - a1_c)
        t1 = jnp.where(o1, w1, 0.0)

        o2 = (a1_r < a1_c) | ((a1_r == a1_c) & tie)
        w2 = jnp.exp2(a2_r - a2_c)
        t2 = jnp.where(o2, w2, 0.0)

        ones = jnp.ones((1, BR), jnp.float32)
        part1 = lax.dot_general(ones, t1, (((1,), (0,)), ((), ())),
                                preferred_element_type=jnp.float32)
        part2 = lax.dot_general(ones, t2, (((1,), (0,)), ((), ())),
                                preferred_element_type=jnp.float32)
        s1_ref[0] += part1
        s2_ref[0] += part2


def kernel(cls, label_cls, pred_bboxes, label_target):
    b = label_cls.shape[0]
    pad = NPAD - N

    cls1 = jnp.pad(cls[:, :, 1], ((0, 0), (0, pad)))
    lab = jnp.pad(label_cls.astype(jnp.int32), ((0, 0), (0, pad)))
    pb = jnp.pad(pred_bboxes, ((0, 0), (0, 0), (0, pad)))
    ltb = jnp.broadcast_to(label_target[:, :, None], (b, 4, 16))

    a1, a2, npos_rows = _sc_compact()(cls1, lab, pb, ltb)
    npos = npos_rows[:, 0]

    a1_c = a1.reshape(b, 1, NPAD)
    a2_c = a2.reshape(b, 1, NPAD)

    def _cmap(i, cb, rb, n):
        return (i, 0, cb)

    def _rmap(i, cb, rb, n):
        return (i, 0, rb)

    col = pl.BlockSpec((1, 1, BC), _cmap)
    row = pl.BlockSpec((1, 1, BR), _rmap)
    out = pl.BlockSpec((1, 1, BC), _cmap)

    grid_spec = pltpu.PrefetchScalarGridSpec(
        num_scalar_prefetch=1,
        grid=(b, NBC, NBR),
        in_specs=[col, col, row, row],
        out_specs=[out, out],
    )
    s1, s2 = pl.pallas_call(
        _pair_body,
        grid_spec=grid_spec,
        out_shape=[
            jax.ShapeDtypeStruct((b, 1, NPAD), jnp.float32),
            jax.ShapeDtypeStruct((b, 1, NPAD), jnp.float32),
        ],
    )(npos, a1_c, a2_c, a1_c, a2_c)

    sum1 = s1.sum(axis=(1, 2))
    sum2 = s2.sum(axis=(1, 2))

    denom = (npos * (npos - 1) // 2).astype(jnp.float32)
    loss1 = sum1 / denom
    loss2 = sum2 / denom
    valid = (npos > 0) & ~jnp.isnan(loss1) & ~jnp.isnan(loss2)
    cnt = valid.sum().astype(jnp.float32)
    any_valid = valid.any()
    safe_cnt = jnp.where(any_valid, cnt, jnp.asarray(1.0, jnp.float32))
    zero = jnp.asarray(0.0, jnp.float32)
    final1 = jnp.where(any_valid, jnp.where(valid, loss1, 0.0).sum() / safe_cnt, zero)
    final2 = jnp.where(any_valid, jnp.where(valid, loss2, 0.0).sum() / safe_cnt, zero)
    return (final1, final2)
